# Initial kernel scaffold; baseline (speedup 1.0000x reference)
import jax
import jax.numpy as jnp
from jax.experimental import pallas as pl


def kernel(inputs, table):
    raise NotImplementedError("write your pallas kernel here")


# speedup vs baseline: 1.4906x; 1.4906x over previous
import functools

import jax
import jax.numpy as jnp
from jax import lax
from jax.experimental import pallas as pl
from jax.experimental.pallas import tpu as pltpu
from jax.experimental.pallas import tpu_sc as plsc

NUM_BINS = 1000000
NUM_HASH = 2
SALTS = (1, 2)
L = 16
CHUNK = 128


def _hash_mod_bins(h):
    hi = plsc.bitcast(h, jnp.int32)
    hf = hi.astype(jnp.float32)
    hf = jnp.where(hi < 0, hf + jnp.float32(4294967296.0), hf)
    q = (hf * jnp.float32(1.0 / NUM_BINS)).astype(jnp.int32)
    r = hi - q * jnp.int32(NUM_BINS)
    r = jnp.where(r < 0, r + jnp.int32(NUM_BINS), r)
    r = jnp.where(r >= jnp.int32(NUM_BINS), r - jnp.int32(NUM_BINS), r)
    return r


def _hash_ids(ids, salt_const, offset):
    h = plsc.bitcast(ids, jnp.uint32)
    h = h * jnp.uint32(2654435761)
    h = h ^ jnp.uint32(salt_const)
    h = h ^ (h >> 16)
    h = h * jnp.uint32(0x85EBCA6B)
    h = h ^ (h >> 13)
    h = h * jnp.uint32(0xC2B2AE35)
    h = h ^ (h >> 16)
    return _hash_mod_bins(h) + jnp.int32(offset)


@functools.partial(jax.jit, static_argnames=("n", "d"))
def _sc_lookup(ids_flat, table, n, d):
    info = plsc.get_sparse_core_info()
    nc, ns = info.num_cores, info.num_subcores
    nw = nc * ns
    per_w = n // nw
    n_chunks = per_w // CHUNK
    mesh = plsc.VectorSubcoreMesh(core_axis_name="c", subcore_axis_name="s")
    salt_consts = [(s * 0x9E3779B9) & 0xFFFFFFFF for s in SALTS]

    @functools.partial(
        pl.kernel,
        mesh=mesh,
        compiler_params=pltpu.CompilerParams(use_tc_tiling_on_sc=False),
        out_type=jax.ShapeDtypeStruct((n, d), jnp.float32),
        scratch_types=[
            pltpu.VMEM((CHUNK,), jnp.int32),
            pltpu.VMEM((CHUNK,), jnp.int32),
            pltpu.VMEM((CHUNK,), jnp.int32),
            pltpu.VMEM((CHUNK, d), jnp.float32),
            pltpu.VMEM((CHUNK, d), jnp.float32),
            pltpu.SemaphoreType.DMA,
        ],
    )
    def k(ids_hbm, table_hbm, out_hbm, ids_v, idx0_v, idx1_v, rows0_v,
          rows1_v, sem):
        wid = lax.axis_index("s") * nc + lax.axis_index("c")
        base = wid * per_w

        def chunk_body(ci, carry):
            off = base + ci * CHUNK
            pltpu.sync_copy(ids_hbm.at[pl.ds(off, CHUNK)], ids_v)
            for j in range(CHUNK // L):
                sl = pl.ds(j * L, L)
                ids = ids_v[sl]
                idx0_v[sl] = _hash_ids(ids, salt_consts[0], 0)
                idx1_v[sl] = _hash_ids(ids, salt_consts[1], NUM_BINS)
            cp0 = pltpu.async_copy(table_hbm.at[idx0_v], rows0_v, sem)
            cp1 = pltpu.async_copy(table_hbm.at[idx1_v], rows1_v, sem)
            cp0.wait()
            cp1.wait()

            def add_body(i, c2):
                for hcol in range(d // L):
                    sl2 = pl.ds(hcol * L, L)
                    rows0_v[i, sl2] = rows0_v[i, sl2] + rows1_v[i, sl2]
                return c2

            lax.fori_loop(0, CHUNK, add_body, 0, unroll=4)
            pltpu.sync_copy(rows0_v, out_hbm.at[pl.ds(off, CHUNK)])
            return carry

        lax.fori_loop(0, n_chunks, chunk_body, 0)

    return k(ids_flat, table)


def kernel(inputs, table):
    b, f = inputs.shape
    d = table.shape[1]
    n = b * f
    ids_flat = inputs.reshape(n)
    info = plsc.get_sparse_core_info()
    grain = info.num_cores * info.num_subcores * CHUNK
    n_pad = ((n + grain - 1) // grain) * grain
    if n_pad != n:
        ids_flat = jnp.pad(ids_flat, (0, n_pad - n))
    out = _sc_lookup(ids_flat, table, n_pad, d)
    return out[:n].reshape(b, f, d)


# speedup vs baseline: 1.6684x; 1.1193x over previous
import functools

import jax
import jax.numpy as jnp
from jax import lax
from jax.experimental import pallas as pl
from jax.experimental.pallas import tpu as pltpu
from jax.experimental.pallas import tpu_sc as plsc

NUM_BINS = 1000000
SALTS = (1, 2)
L = 16
IW = 128
K = 2
CHUNK = IW * K


def _hash_mod_bins(h):
    hi = plsc.bitcast(h, jnp.int32)
    hf = hi.astype(jnp.float32)
    hf = jnp.where(hi < 0, hf + jnp.float32(4294967296.0), hf)
    q = (hf * jnp.float32(1.0 / NUM_BINS)).astype(jnp.int32)
    r = hi - q * jnp.int32(NUM_BINS)
    r = jnp.where(r < 0, r + jnp.int32(NUM_BINS), r)
    r = jnp.where(r >= jnp.int32(NUM_BINS), r - jnp.int32(NUM_BINS), r)
    return r


def _hash_ids(ids, salt_const, offset):
    h = plsc.bitcast(ids, jnp.uint32)
    h = h * jnp.uint32(2654435761)
    h = h ^ jnp.uint32(salt_const)
    h = h ^ (h >> 16)
    h = h * jnp.uint32(0x85EBCA6B)
    h = h ^ (h >> 13)
    h = h * jnp.uint32(0xC2B2AE35)
    h = h ^ (h >> 16)
    return _hash_mod_bins(h) + jnp.int32(offset)


@functools.partial(jax.jit, static_argnames=("n", "d"))
def _sc_lookup(ids_flat, table, n, d):
    info = plsc.get_sparse_core_info()
    nc, ns = info.num_cores, info.num_subcores
    nw = nc * ns
    per_w = n // nw
    n_chunks = per_w // CHUNK
    n_half = n_chunks // 2
    mesh = plsc.VectorSubcoreMesh(core_axis_name="c", subcore_axis_name="s")
    salt_consts = [(s * 0x9E3779B9) & 0xFFFFFFFF for s in SALTS]

    @functools.partial(
        pl.kernel,
        mesh=mesh,
        compiler_params=pltpu.CompilerParams(use_tc_tiling_on_sc=False),
        out_type=jax.ShapeDtypeStruct((n, d), jnp.float32),
        scratch_types=[
            pltpu.VMEM((CHUNK,), jnp.int32),
            pltpu.VMEM((CHUNK,), jnp.int32),
            pltpu.VMEM((2 * K, IW), jnp.int32),
            pltpu.VMEM((2 * K, IW), jnp.int32),
            pltpu.VMEM((CHUNK, d), jnp.float32),
            pltpu.VMEM((CHUNK, d), jnp.float32),
            pltpu.VMEM((CHUNK, d), jnp.float32),
            pltpu.VMEM((CHUNK, d), jnp.float32),
            pltpu.SemaphoreType.DMA,
            pltpu.SemaphoreType.DMA,
            pltpu.SemaphoreType.DMA,
            pltpu.SemaphoreType.DMA,
            pltpu.SemaphoreType.DMA,
            pltpu.SemaphoreType.DMA,
        ],
    )
    def k(ids_hbm, table_hbm, out_hbm, ids_v0, ids_v1, idx_v0, idx_v1,
          r0_v0, r1_v0, r0_v1, r1_v1, si0, si1, sg0, sg1, so0, so1):
        wid = lax.axis_index("s") * nc + lax.axis_index("c")
        base = wid * per_w
        ids_v = (ids_v0, ids_v1)
        idx_v = (idx_v0, idx_v1)
        r_v = ((r0_v0, r1_v0), (r0_v1, r1_v1))
        sem_i = (si0, si1)
        sem_g = (sg0, sg1)
        sem_o = (so0, so1)

        def hash_chunk(b):
            for j in range(CHUNK // L):
                row, col = (j * L) // IW, (j * L) % IW
                ids = ids_v[b][pl.ds(j * L, L)]
                idx_v[b][row, pl.ds(col, L)] = _hash_ids(ids, salt_consts[0], 0)
                idx_v[b][K + row, pl.ds(col, L)] = _hash_ids(
                    ids, salt_consts[1], NUM_BINS)

        def gather_cps(b):
            cps = []
            for h in range(2):
                for kk in range(K):
                    cps.append(pltpu.make_async_copy(
                        table_hbm.at[idx_v[b].at[h * K + kk]],
                        r_v[b][h].at[pl.ds(kk * IW, IW)],
                        sem_g[b]))
            return cps

        def out_cp(b, ci):
            return pltpu.make_async_copy(
                r_v[b][0], out_hbm.at[pl.ds(base + ci * CHUNK, CHUNK)],
                sem_o[b])

        def add_rows(b):
            def add_body(i, c2):
                for hcol in range(d // L):
                    sl2 = pl.ds(hcol * L, L)
                    r_v[b][0][i, sl2] = r_v[b][0][i, sl2] + r_v[b][1][i, sl2]
                return c2
            lax.fori_loop(0, CHUNK, add_body, 0, unroll=8)

        pltpu.sync_copy(ids_hbm.at[pl.ds(base, CHUNK)], ids_v[0])
        pltpu.async_copy(ids_hbm.at[pl.ds(base + CHUNK, CHUNK)], ids_v[1],
                         sem_i[1])
        hash_chunk(0)
        for cp in gather_cps(0):
            cp.start()

        def body(i, carry):
            for b in (0, 1):
                ci = 2 * i + b
                other = 1 - b
                for cp in gather_cps(b):
                    cp.wait()

                @pl.when(ci + 1 < n_chunks)
                def _stage_next():
                    pltpu.make_async_copy(
                        ids_hbm.at[pl.ds(base + (ci + 1) * CHUNK, CHUNK)],
                        ids_v[other], sem_i[other]).wait()
                    hash_chunk(other)

                    @pl.when(ci >= 1)
                    def _drain_prev_store():
                        out_cp(other, ci).wait()

                    for cp in gather_cps(other):
                        cp.start()

                    @pl.when(ci + 2 < n_chunks)
                    def _prefetch_ids():
                        pltpu.async_copy(
                            ids_hbm.at[pl.ds(base + (ci + 2) * CHUNK, CHUNK)],
                            ids_v[b], sem_i[b])

                add_rows(b)
                out_cp(b, ci).start()
            return carry

        lax.fori_loop(0, n_half, body, 0)
        out_cp(0, 0).wait()
        out_cp(1, 0).wait()

    return k(ids_flat, table)


def kernel(inputs, table):
    b, f = inputs.shape
    d = table.shape[1]
    n = b * f
    ids_flat = inputs.reshape(n)
    info = plsc.get_sparse_core_info()
    grain = info.num_cores * info.num_subcores * CHUNK * 2
    n_pad = ((n + grain - 1) // grain) * grain
    if n_pad != n:
        ids_flat = jnp.pad(ids_flat, (0, n_pad - n))
    out = _sc_lookup(ids_flat, table, n_pad, d)
    return out[:n].reshape(b, f, d)
